# baseline (speedup 1.0000x reference)
import jax
import jax.numpy as jnp
from jax import lax
from jax.experimental import pallas as pl
from jax.experimental.pallas import tpu as pltpu
from jax.experimental.pallas import tpu_sc as plsc

NUM_EMBEDDINGS = 1000000
EMBEDDING_DIM = 64
BATCH = 16384
N_FIELDS = 26

NC = 2
NS = 16
NW = NC * NS
LANES = 16

B_TOTAL = BATCH * N_FIELDS
CHUNK = 128
N_BLOCKS = B_TOTAL // CHUNK
BLK_PER_W = N_BLOCKS // NW
NBUF = 4
K = 3
NOBUF = 2
BBLKS = BATCH // CHUNK


def _body(ids_hbm, table_hbm, out_hbm, idx_v, rows_v, tr_v, in_sems, out_sems):
    wid = lax.axis_index("s") * NC + lax.axis_index("c")
    g0 = wid * BLK_PER_W

    pltpu.sync_copy(ids_hbm.at[wid], idx_v)

    def issue_gather(i, buf):
        pltpu.async_copy(table_hbm.at[idx_v.at[i]], rows_v.at[buf],
                         in_sems.at[buf])

    def wait_gather(i, buf):
        pltpu.make_async_copy(table_hbm.at[idx_v.at[i]], rows_v.at[buf],
                              in_sems.at[buf]).wait()

    def out_slice(i, obuf):
        g = g0 + i
        f = g // BBLKS
        b0 = (g % BBLKS) * CHUNK
        return tr_v.at[obuf], out_hbm.at[f, :, pl.ds(b0, CHUNK)]

    def issue_store(i, obuf):
        src, dst = out_slice(i, obuf)
        pltpu.async_copy(src, dst, out_sems.at[obuf])

    def wait_store(i, obuf):
        src, dst = out_slice(i, obuf)
        pltpu.make_async_copy(src, dst, out_sems.at[obuf]).wait()

    def transpose_block(buf, obuf):
        rows = rows_v.at[buf]
        tr = tr_v.at[obuf]

        def d_body(d, _):
            dv = jnp.full((LANES,), 0, jnp.int32) + d
            for j0 in range(CHUNK // LANES):
                jv = lax.iota(jnp.int32, LANES) + (j0 * LANES)
                x = plsc.load_gather(rows, [jv, dv])
                tr[d, pl.ds(j0 * LANES, LANES)] = x
            return 0

        lax.fori_loop(0, EMBEDDING_DIM, d_body, 0, unroll=2)

    def step(i, bi, *, head, tail):
        obuf = bi % NOBUF
        if not tail:
            issue_gather(i + K, (bi + K) % NBUF)
        wait_gather(i, bi)
        if not head:
            wait_store(i - NOBUF, obuf)
        transpose_block(bi, obuf)
        issue_store(i, obuf)

    for i in range(K):
        issue_gather(i, i)
    for bi in range(NBUF):
        step(bi, bi, head=(bi < NOBUF), tail=False)

    def loop_body(g, _):
        i0 = g * NBUF
        for bi in range(NBUF):
            step(i0 + bi, bi, head=False, tail=False)
        return 0

    lax.fori_loop(1, BLK_PER_W // NBUF - 1, loop_body, 0, unroll=False)

    i0 = BLK_PER_W - NBUF
    for bi in range(NBUF):
        step(i0 + bi, bi, head=False, tail=(bi + K >= NBUF))

    for i in range(BLK_PER_W - NOBUF, BLK_PER_W):
        wait_store(i, i % NOBUF)


@jax.jit
def _gather(ids_grouped, table):
    mesh = plsc.VectorSubcoreMesh(core_axis_name="c", subcore_axis_name="s",
                                  num_cores=NC, num_subcores=NS)
    f = pl.kernel(
        _body,
        out_type=jax.ShapeDtypeStruct((N_FIELDS, EMBEDDING_DIM, BATCH),
                                      jnp.float32),
        mesh=mesh,
        scratch_types=[
            pltpu.VMEM((BLK_PER_W, CHUNK), jnp.int32),
            pltpu.VMEM((NBUF, CHUNK, EMBEDDING_DIM), jnp.float32),
            pltpu.VMEM((NOBUF, EMBEDDING_DIM, CHUNK), jnp.float32),
            pltpu.SemaphoreType.DMA((NBUF,)),
            pltpu.SemaphoreType.DMA((NOBUF,)),
        ],
        compiler_params=pltpu.CompilerParams(use_tc_tiling_on_sc=False,
                                             needs_layout_passes=False),
    )
    return f(ids_grouped, table)


def kernel(ids, table):
    ids_grouped = ids.T.reshape(NW, BLK_PER_W, CHUNK).astype(jnp.int32)
    out_phys = _gather(ids_grouped, table)
    return out_phys.transpose(2, 0, 1)


# speedup vs baseline: 1.0955x; 1.0955x over previous
import jax
import jax.numpy as jnp
from jax import lax
from jax.experimental import pallas as pl
from jax.experimental.pallas import tpu as pltpu
from jax.experimental.pallas import tpu_sc as plsc

NUM_EMBEDDINGS = 1000000
EMBEDDING_DIM = 64
BATCH = 16384
N_FIELDS = 26

NC = 2
NS = 16
NW = NC * NS
LANES = 16

B_TOTAL = BATCH * N_FIELDS
CHUNK = 128
N_BLOCKS = B_TOTAL // CHUNK
BLK_PER_W = N_BLOCKS // NW
NBUF = 4
K = 3
NOBUF = 2
BBLKS = BATCH // CHUNK


def _body(ids_hbm, table_hbm, out_hbm, idx_v, rows_v, tr_v, in_sems, out_sems):
    wid = lax.axis_index("s") * NC + lax.axis_index("c")
    g0 = wid * BLK_PER_W

    pltpu.sync_copy(ids_hbm.at[wid], idx_v)

    def issue_gather(i, buf):
        pltpu.async_copy(table_hbm.at[idx_v.at[i]], rows_v.at[buf],
                         in_sems.at[buf])

    def wait_gather(i, buf):
        pltpu.make_async_copy(table_hbm.at[idx_v.at[i]], rows_v.at[buf],
                              in_sems.at[buf]).wait()

    def out_slice(i, obuf):
        g = g0 + i
        f = g // BBLKS
        b0 = (g % BBLKS) * CHUNK
        return tr_v.at[obuf], out_hbm.at[f, :, pl.ds(b0, CHUNK)]

    def issue_store(i, obuf):
        src, dst = out_slice(i, obuf)
        pltpu.async_copy(src, dst, out_sems.at[obuf])

    def wait_store(i, obuf):
        src, dst = out_slice(i, obuf)
        pltpu.make_async_copy(src, dst, out_sems.at[obuf]).wait()

    def transpose_block(buf, obuf):
        rows = rows_v.at[buf]
        tr = tr_v.at[obuf]

        def d_body(d, _):
            dv = jnp.full((LANES,), 0, jnp.int32) + d
            xs = []
            for j0 in range(CHUNK // LANES):
                jv = lax.iota(jnp.int32, LANES) + (j0 * LANES)
                xs.append(plsc.load_gather(rows, [jv, dv]))
            for j0 in range(CHUNK // LANES):
                tr[d, pl.ds(j0 * LANES, LANES)] = xs[j0]
            return 0

        lax.fori_loop(0, EMBEDDING_DIM, d_body, 0, unroll=4)

    def step(i, bi, *, head, tail):
        obuf = bi % NOBUF
        if not tail:
            issue_gather(i + K, (bi + K) % NBUF)
        wait_gather(i, bi)
        if not head:
            wait_store(i - NOBUF, obuf)
        transpose_block(bi, obuf)
        issue_store(i, obuf)

    for i in range(K):
        issue_gather(i, i)
    for bi in range(NBUF):
        step(bi, bi, head=(bi < NOBUF), tail=False)

    def loop_body(g, _):
        i0 = g * NBUF
        for bi in range(NBUF):
            step(i0 + bi, bi, head=False, tail=False)
        return 0

    lax.fori_loop(1, BLK_PER_W // NBUF - 1, loop_body, 0, unroll=False)

    i0 = BLK_PER_W - NBUF
    for bi in range(NBUF):
        step(i0 + bi, bi, head=False, tail=(bi + K >= NBUF))

    for i in range(BLK_PER_W - NOBUF, BLK_PER_W):
        wait_store(i, i % NOBUF)


@jax.jit
def _gather(ids_grouped, table):
    mesh = plsc.VectorSubcoreMesh(core_axis_name="c", subcore_axis_name="s",
                                  num_cores=NC, num_subcores=NS)
    f = pl.kernel(
        _body,
        out_type=jax.ShapeDtypeStruct((N_FIELDS, EMBEDDING_DIM, BATCH),
                                      jnp.float32),
        mesh=mesh,
        scratch_types=[
            pltpu.VMEM((BLK_PER_W, CHUNK), jnp.int32),
            pltpu.VMEM((NBUF, CHUNK, EMBEDDING_DIM), jnp.float32),
            pltpu.VMEM((NOBUF, EMBEDDING_DIM, CHUNK), jnp.float32),
            pltpu.SemaphoreType.DMA((NBUF,)),
            pltpu.SemaphoreType.DMA((NOBUF,)),
        ],
        compiler_params=pltpu.CompilerParams(use_tc_tiling_on_sc=False,
                                             needs_layout_passes=False),
    )
    return f(ids_grouped, table)


def kernel(ids, table):
    ids_grouped = ids.T.reshape(NW, BLK_PER_W, CHUNK).astype(jnp.int32)
    out_phys = _gather(ids_grouped, table)
    return out_phys.transpose(2, 0, 1)


# speedup vs baseline: 1.1417x; 1.0422x over previous
import jax
import jax.numpy as jnp
from jax import lax
from jax.experimental import pallas as pl
from jax.experimental.pallas import tpu as pltpu
from jax.experimental.pallas import tpu_sc as plsc

NUM_EMBEDDINGS = 1000000
EMBEDDING_DIM = 64
BATCH = 16384
N_FIELDS = 26

NC = 2
NS = 16
NW = NC * NS
LANES = 16

B_TOTAL = BATCH * N_FIELDS
CHUNK = 128
N_BLOCKS = B_TOTAL // CHUNK
BLK_PER_W = N_BLOCKS // NW
NBUF = 4
K = 3
NOBUF = 2
BBLKS = BATCH // CHUNK


def _body(ids_hbm, table_hbm, out_hbm, idx_v, rows_v, tr_v, in_sems, out_sems):
    wid = lax.axis_index("s") * NC + lax.axis_index("c")
    g0 = wid * BLK_PER_W

    pltpu.sync_copy(ids_hbm.at[wid], idx_v)

    def issue_gather(i, buf):
        pltpu.async_copy(table_hbm.at[idx_v.at[i]], rows_v.at[buf],
                         in_sems.at[buf])

    def wait_gather(i, buf):
        pltpu.make_async_copy(table_hbm.at[idx_v.at[i]], rows_v.at[buf],
                              in_sems.at[buf]).wait()

    def out_slice(i, obuf):
        g = g0 + i
        f = g // BBLKS
        b0 = (g % BBLKS) * CHUNK
        return tr_v.at[obuf], out_hbm.at[f, :, pl.ds(b0, CHUNK)]

    def issue_store(i, obuf):
        src, dst = out_slice(i, obuf)
        pltpu.async_copy(src, dst, out_sems.at[obuf])

    def wait_store(i, obuf):
        src, dst = out_slice(i, obuf)
        pltpu.make_async_copy(src, dst, out_sems.at[obuf]).wait()

    def transpose_block(buf, obuf):
        rows = rows_v.at[buf]
        tr = tr_v.at[obuf]

        @plsc.parallel_loop(0, EMBEDDING_DIM, unroll=4)
        def d_body(d):
            dv = jnp.full((LANES,), 0, jnp.int32) + d
            xs = []
            for j0 in range(CHUNK // LANES):
                jv = lax.iota(jnp.int32, LANES) + (j0 * LANES)
                xs.append(plsc.load_gather(rows, [jv, dv]))
            for j0 in range(CHUNK // LANES):
                tr[d, pl.ds(j0 * LANES, LANES)] = xs[j0]

    def step(i, bi, *, head, tail):
        obuf = bi % NOBUF
        if not tail:
            issue_gather(i + K, (bi + K) % NBUF)
        wait_gather(i, bi)
        if not head:
            wait_store(i - NOBUF, obuf)
        transpose_block(bi, obuf)
        issue_store(i, obuf)

    for i in range(K):
        issue_gather(i, i)
    for bi in range(NBUF):
        step(bi, bi, head=(bi < NOBUF), tail=False)

    def loop_body(g, _):
        i0 = g * NBUF
        for bi in range(NBUF):
            step(i0 + bi, bi, head=False, tail=False)
        return 0

    lax.fori_loop(1, BLK_PER_W // NBUF - 1, loop_body, 0, unroll=False)

    i0 = BLK_PER_W - NBUF
    for bi in range(NBUF):
        step(i0 + bi, bi, head=False, tail=(bi + K >= NBUF))

    for i in range(BLK_PER_W - NOBUF, BLK_PER_W):
        wait_store(i, i % NOBUF)


@jax.jit
def _gather(ids_grouped, table):
    mesh = plsc.VectorSubcoreMesh(core_axis_name="c", subcore_axis_name="s",
                                  num_cores=NC, num_subcores=NS)
    f = pl.kernel(
        _body,
        out_type=jax.ShapeDtypeStruct((N_FIELDS, EMBEDDING_DIM, BATCH),
                                      jnp.float32),
        mesh=mesh,
        scratch_types=[
            pltpu.VMEM((BLK_PER_W, CHUNK), jnp.int32),
            pltpu.VMEM((NBUF, CHUNK, EMBEDDING_DIM), jnp.float32),
            pltpu.VMEM((NOBUF, EMBEDDING_DIM, CHUNK), jnp.float32),
            pltpu.SemaphoreType.DMA((NBUF,)),
            pltpu.SemaphoreType.DMA((NOBUF,)),
        ],
        compiler_params=pltpu.CompilerParams(use_tc_tiling_on_sc=False,
                                             needs_layout_passes=False,
                                             disable_bounds_checks=True),
    )
    return f(ids_grouped, table)


def kernel(ids, table):
    ids_grouped = ids.T.reshape(NW, BLK_PER_W, CHUNK).astype(jnp.int32)
    out_phys = _gather(ids_grouped, table)
    return out_phys.transpose(2, 0, 1)


# speedup vs baseline: 1.4483x; 1.2686x over previous
import jax
import jax.numpy as jnp
from jax import lax
from jax.experimental import pallas as pl
from jax.experimental.pallas import tpu as pltpu
from jax.experimental.pallas import tpu_sc as plsc

NUM_EMBEDDINGS = 1000000
EMBEDDING_DIM = 64
BATCH = 16384
N_FIELDS = 26

NC = 2
NS = 16
NW = NC * NS
LANES = 16

B_TOTAL = BATCH * N_FIELDS
CHUNK = 128
N_BLOCKS = B_TOTAL // CHUNK
BLK_PER_W = N_BLOCKS // NW
NBUF = 4
K = 3
NOBUF = 2
BBLKS = BATCH // CHUNK


def _body(ids_hbm, table_hbm, out_hbm, idx_v, rows_v, tr_v, in_sems, out_sems):
    wid = lax.axis_index("s") * NC + lax.axis_index("c")
    g0 = wid * BLK_PER_W

    pltpu.sync_copy(ids_hbm.at[wid], idx_v)

    def issue_gather(i, buf):
        pltpu.async_copy(table_hbm.at[idx_v.at[i]], rows_v.at[buf],
                         in_sems.at[buf])

    def wait_gather(i, buf):
        pltpu.make_async_copy(table_hbm.at[idx_v.at[i]], rows_v.at[buf],
                              in_sems.at[buf]).wait()

    def out_slice(i, obuf):
        g = g0 + i
        f = g // BBLKS
        b0 = (g % BBLKS) * CHUNK
        return tr_v.at[obuf], out_hbm.at[f, :, pl.ds(b0, CHUNK)]

    def issue_store(i, obuf):
        src, dst = out_slice(i, obuf)
        pltpu.async_copy(src, dst, out_sems.at[obuf])

    def wait_store(i, obuf):
        src, dst = out_slice(i, obuf)
        pltpu.make_async_copy(src, dst, out_sems.at[obuf]).wait()

    def transpose_block(buf, obuf):
        rows = rows_v.at[buf]
        tr = tr_v.at[obuf]

        iv = lax.iota(jnp.int32, LANES)

        def d_body(dt, _):
            d0 = dt * LANES
            for t in range(LANES):
                dvb = ((iv + t) & (LANES - 1)) + d0
                for j0 in range(CHUNK // LANES):
                    jv = iv + (j0 * LANES)
                    x = plsc.load_gather(rows, [jv, dvb])
                    plsc.store_scatter(tr, [dvb, jv], x)
            return 0

        lax.fori_loop(0, EMBEDDING_DIM // LANES, d_body, 0, unroll=False)

    def step(i, bi, *, head, tail):
        obuf = bi % NOBUF
        if not tail:
            issue_gather(i + K, (bi + K) % NBUF)
        wait_gather(i, bi)
        if not head:
            wait_store(i - NOBUF, obuf)
        transpose_block(bi, obuf)
        issue_store(i, obuf)

    for i in range(K):
        issue_gather(i, i)
    for bi in range(NBUF):
        step(bi, bi, head=(bi < NOBUF), tail=False)

    def loop_body(g, _):
        i0 = g * NBUF
        for bi in range(NBUF):
            step(i0 + bi, bi, head=False, tail=False)
        return 0

    lax.fori_loop(1, BLK_PER_W // NBUF - 1, loop_body, 0, unroll=False)

    i0 = BLK_PER_W - NBUF
    for bi in range(NBUF):
        step(i0 + bi, bi, head=False, tail=(bi + K >= NBUF))

    for i in range(BLK_PER_W - NOBUF, BLK_PER_W):
        wait_store(i, i % NOBUF)


@jax.jit
def _gather(ids_grouped, table):
    mesh = plsc.VectorSubcoreMesh(core_axis_name="c", subcore_axis_name="s",
                                  num_cores=NC, num_subcores=NS)
    f = pl.kernel(
        _body,
        out_type=jax.ShapeDtypeStruct((N_FIELDS, EMBEDDING_DIM, BATCH),
                                      jnp.float32),
        mesh=mesh,
        scratch_types=[
            pltpu.VMEM((BLK_PER_W, CHUNK), jnp.int32),
            pltpu.VMEM((NBUF, CHUNK, EMBEDDING_DIM), jnp.float32),
            pltpu.VMEM((NOBUF, EMBEDDING_DIM, CHUNK), jnp.float32),
            pltpu.SemaphoreType.DMA((NBUF,)),
            pltpu.SemaphoreType.DMA((NOBUF,)),
        ],
        compiler_params=pltpu.CompilerParams(use_tc_tiling_on_sc=False,
                                             needs_layout_passes=False,
                                             disable_bounds_checks=True),
    )
    return f(ids_grouped, table)


def kernel(ids, table):
    ids_grouped = ids.T.reshape(NW, BLK_PER_W, CHUNK).astype(jnp.int32)
    out_phys = _gather(ids_grouped, table)
    return out_phys.transpose(2, 0, 1)


# speedup vs baseline: 1.6660x; 1.1503x over previous
import jax
import jax.numpy as jnp
from jax import lax
from jax.experimental import pallas as pl
from jax.experimental.pallas import tpu as pltpu
from jax.experimental.pallas import tpu_sc as plsc

NUM_EMBEDDINGS = 1000000
EMBEDDING_DIM = 64
BATCH = 16384
N_FIELDS = 26

NC = 2
NS = 16
NW = NC * NS
LANES = 16

B_TOTAL = BATCH * N_FIELDS
CHUNK = 128
N_BLOCKS = B_TOTAL // CHUNK
BLK_PER_W = N_BLOCKS // NW
NBUF = 4
K = 3
NOBUF = 2
BBLKS = BATCH // CHUNK


def _body(ids_hbm, table_hbm, out_hbm, idx_v, rows_v, tr_v, in_sems, out_sems):
    wid = lax.axis_index("s") * NC + lax.axis_index("c")
    g0 = wid * BLK_PER_W

    pltpu.sync_copy(ids_hbm.at[wid], idx_v)

    def issue_gather(i, buf):
        pltpu.async_copy(table_hbm.at[idx_v.at[i]], rows_v.at[buf],
                         in_sems.at[buf])

    def wait_gather(i, buf):
        pltpu.make_async_copy(table_hbm.at[idx_v.at[i]], rows_v.at[buf],
                              in_sems.at[buf]).wait()

    def out_slice(i, obuf):
        g = g0 + i
        f = g // BBLKS
        b0 = (g % BBLKS) * CHUNK
        return tr_v.at[obuf], out_hbm.at[f, :, pl.ds(b0, CHUNK)]

    def issue_store(i, obuf):
        src, dst = out_slice(i, obuf)
        pltpu.async_copy(src, dst, out_sems.at[obuf])

    def wait_store(i, obuf):
        src, dst = out_slice(i, obuf)
        pltpu.make_async_copy(src, dst, out_sems.at[obuf]).wait()

    def transpose_block(buf, obuf):
        rows = rows_v.at[buf]
        tr = tr_v.at[obuf]

        iv = lax.iota(jnp.int32, LANES)

        def d_body(dt, _):
            d0 = dt * LANES
            for t in range(LANES):
                dvb = ((iv + t) & (LANES - 1)) + d0
                xs = []
                for j0 in range(CHUNK // LANES):
                    jv = iv + (j0 * LANES)
                    xs.append(plsc.load_gather(rows, [jv, dvb]))
                for j0 in range(CHUNK // LANES):
                    jv = iv + (j0 * LANES)
                    plsc.store_scatter(tr, [dvb, jv], xs[j0])
            return 0

        lax.fori_loop(0, EMBEDDING_DIM // LANES, d_body, 0, unroll=False)

    def step(i, bi, *, head, tail):
        obuf = bi % NOBUF
        if not tail:
            issue_gather(i + K, (bi + K) % NBUF)
        wait_gather(i, bi)
        if not head:
            wait_store(i - NOBUF, obuf)
        transpose_block(bi, obuf)
        issue_store(i, obuf)

    for i in range(K):
        issue_gather(i, i)
    for bi in range(NBUF):
        step(bi, bi, head=(bi < NOBUF), tail=False)

    def loop_body(g, _):
        i0 = g * NBUF
        for bi in range(NBUF):
            step(i0 + bi, bi, head=False, tail=False)
        return 0

    lax.fori_loop(1, BLK_PER_W // NBUF - 1, loop_body, 0, unroll=False)

    i0 = BLK_PER_W - NBUF
    for bi in range(NBUF):
        step(i0 + bi, bi, head=False, tail=(bi + K >= NBUF))

    for i in range(BLK_PER_W - NOBUF, BLK_PER_W):
        wait_store(i, i % NOBUF)


@jax.jit
def _gather(ids_grouped, table):
    mesh = plsc.VectorSubcoreMesh(core_axis_name="c", subcore_axis_name="s",
                                  num_cores=NC, num_subcores=NS)
    f = pl.kernel(
        _body,
        out_type=jax.ShapeDtypeStruct((N_FIELDS, EMBEDDING_DIM, BATCH),
                                      jnp.float32),
        mesh=mesh,
        scratch_types=[
            pltpu.VMEM((BLK_PER_W, CHUNK), jnp.int32),
            pltpu.VMEM((NBUF, CHUNK, EMBEDDING_DIM), jnp.float32),
            pltpu.VMEM((NOBUF, EMBEDDING_DIM, CHUNK), jnp.float32),
            pltpu.SemaphoreType.DMA((NBUF,)),
            pltpu.SemaphoreType.DMA((NOBUF,)),
        ],
        compiler_params=pltpu.CompilerParams(use_tc_tiling_on_sc=False,
                                             needs_layout_passes=False,
                                             disable_bounds_checks=True),
    )
    return f(ids_grouped, table)


def kernel(ids, table):
    ids_grouped = ids.T.reshape(NW, BLK_PER_W, CHUNK).astype(jnp.int32)
    out_phys = _gather(ids_grouped, table)
    return out_phys.transpose(2, 0, 1)
